# baseline (speedup 1.0000x reference)
import numpy as np

import jax
import jax.numpy as jnp
from jax.experimental import pallas as pl

_N = 4096
_D = 256
_ALPHA = 3.0
_TOPK = 16
_BLOCK = 256
_SEG = 128
_NSEG = _N // _SEG
_INT_MIN = jnp.iinfo(jnp.int32).min

_UT128 = np.triu(np.ones((_SEG, _SEG), np.float32))
_SUT32 = np.triu(np.ones((_NSEG, _NSEG), np.float32), 1)


def _embed_kernel(x1, w1, b1, x2, w2, b2, e1_out, e2_out):
    z1 = jax.lax.dot_general(x1[...], w1[...], (((1,), (1,)), ((), ())),
                             preferred_element_type=jnp.float32)
    e1_out[...] = jnp.tanh(_ALPHA * (z1 + b1[...]))
    z2 = jax.lax.dot_general(x2[...], w2[...], (((1,), (1,)), ((), ())),
                             preferred_element_type=jnp.float32)
    e2_out[...] = jnp.tanh(_ALPHA * (z2 + b2[...]))


def _adj_kernel(e1b, e2b, e1, e2, ut128, sut32, out):
    a = jax.lax.dot_general(e1b[...], e2[...], (((1,), (1,)), ((), ())),
                            preferred_element_type=jnp.float32)
    b = jax.lax.dot_general(e2b[...], e1[...], (((1,), (1,)), ((), ())),
                            preferred_element_type=jnp.float32)
    adj = jnp.maximum(jnp.tanh(_ALPHA * (a - b)), 0.0)

    bits = jax.lax.bitcast_convert_type(adj, jnp.int32)
    sb = bits & ~0xFFF

    vmax = jnp.max(sb, axis=1, keepdims=True)
    condf = (sb == vmax).astype(jnp.float32)

    pieces = []
    for s in range(_NSEG):
        sl = condf[:, s * _SEG:(s + 1) * _SEG]
        pieces.append(jax.lax.dot_general(
            sl, ut128[...], (((1,), (0,)), ((), ())),
            preferred_element_type=jnp.float32))
    cnts = jnp.concatenate([p[:, _SEG - 1:_SEG] for p in pieces], axis=1)
    excl = jax.lax.dot_general(cnts, sut32[...], (((1,), (0,)), ((), ())),
                               preferred_element_type=jnp.float32)
    total = excl[:, _NSEG - 1:_NSEG] + cnts[:, _NSEG - 1:_NSEG]
    ok = jnp.all(total >= float(_TOPK))

    thr = float(_TOPK) - excl
    for s in range(_NSEG):
        sl = slice(s * _SEG, (s + 1) * _SEG)
        m_s = (condf[:, sl] > 0.0) & (pieces[s] <= thr[:, s:s + 1])
        out[:, sl] = jnp.where(m_s, adj[:, sl], 0.0)

    @pl.when(jnp.logical_not(ok))
    def _slow():
        col = jax.lax.broadcasted_iota(jnp.int32, adj.shape, 1)
        keys = sb | ((_N - 1) - col)
        k = keys
        m = None
        for _ in range(_TOPK):
            m = jnp.max(k, axis=1, keepdims=True)
            k = jnp.where(k == m, _INT_MIN, k)
        out[...] = jnp.where(keys >= m, adj, 0.0)


def kernel(idx, emb1_w, emb2_w, theta1_w, theta1_b, theta2_w, theta2_b):
    x1 = jnp.take(emb1_w, idx, axis=0)
    x2 = jnp.take(emb2_w, idx, axis=0)
    e1, e2 = pl.pallas_call(
        _embed_kernel,
        out_shape=[jax.ShapeDtypeStruct((_N, _D), jnp.float32)] * 2,
    )(x1, theta1_w, theta1_b.reshape(1, _D), x2, theta2_w, theta2_b.reshape(1, _D))

    grid = (_N // _BLOCK,)
    out = pl.pallas_call(
        _adj_kernel,
        grid=grid,
        in_specs=[
            pl.BlockSpec((_BLOCK, _D), lambda i: (i, 0)),
            pl.BlockSpec((_BLOCK, _D), lambda i: (i, 0)),
            pl.BlockSpec((_N, _D), lambda i: (0, 0)),
            pl.BlockSpec((_N, _D), lambda i: (0, 0)),
            pl.BlockSpec((_SEG, _SEG), lambda i: (0, 0)),
            pl.BlockSpec((_NSEG, _NSEG), lambda i: (0, 0)),
        ],
        out_specs=pl.BlockSpec((_BLOCK, _N), lambda i: (i, 0)),
        out_shape=jax.ShapeDtypeStruct((_N, _N), jnp.float32),
    )(e1, e2, e1, e2, jnp.asarray(_UT128), jnp.asarray(_SUT32))
    return out


# speedup vs baseline: 1.0472x; 1.0472x over previous
import numpy as np

import jax
import jax.numpy as jnp
from jax.experimental import pallas as pl

_N = 4096
_D = 256
_ALPHA = 3.0
_TOPK = 16
_BLOCK = 256
_SEG = 128
_NSEG = _N // _SEG
_INT_MIN = jnp.iinfo(jnp.int32).min

_UT128 = np.triu(np.ones((_SEG, _SEG), np.float32))
_SUT32 = np.triu(np.ones((_NSEG, _NSEG), np.float32), 1)
_EMAT = (np.arange(_N)[:, None] // _SEG ==
         np.arange(_NSEG)[None, :]).astype(np.float32)


def _embed_kernel(x1, w1, b1, x2, w2, b2, e1_out, e2_out):
    z1 = jax.lax.dot_general(x1[...], w1[...], (((1,), (1,)), ((), ())),
                             preferred_element_type=jnp.float32)
    e1_out[...] = jnp.tanh(_ALPHA * (z1 + b1[...]))
    z2 = jax.lax.dot_general(x2[...], w2[...], (((1,), (1,)), ((), ())),
                             preferred_element_type=jnp.float32)
    e2_out[...] = jnp.tanh(_ALPHA * (z2 + b2[...]))


def _adj_kernel(e1b, e2b, e1, e2, ut128, sut32, emat, out):
    a = jax.lax.dot_general(e1b[...], e2[...], (((1,), (1,)), ((), ())),
                            preferred_element_type=jnp.float32)
    b = jax.lax.dot_general(e2b[...], e1[...], (((1,), (1,)), ((), ())),
                            preferred_element_type=jnp.float32)
    adj = jnp.maximum(jnp.tanh(_ALPHA * (a - b)), 0.0)

    bits = jax.lax.bitcast_convert_type(adj, jnp.int32)
    sb = bits & ~0xFFF

    vmax = jnp.max(sb, axis=1, keepdims=True)
    condf = (sb == vmax).astype(jnp.float32)

    pieces = []
    for s in range(_NSEG):
        sl = condf[:, s * _SEG:(s + 1) * _SEG]
        pieces.append(jax.lax.dot_general(
            sl, ut128[...], (((1,), (0,)), ((), ())),
            preferred_element_type=jnp.float32))
    wcs = jnp.concatenate(pieces, axis=1)
    cnts = jnp.concatenate([p[:, _SEG - 1:_SEG] for p in pieces], axis=1)
    excl = jax.lax.dot_general(cnts, sut32[...], (((1,), (0,)), ((), ())),
                               preferred_element_type=jnp.float32)
    base = jax.lax.dot_general(excl, emat[...], (((1,), (1,)), ((), ())),
                               preferred_element_type=jnp.float32)
    total = excl[:, _NSEG - 1:_NSEG] + cnts[:, _NSEG - 1:_NSEG]
    ok = jnp.all(total >= float(_TOPK))

    mask_fast = (condf > 0.0) & (wcs + base <= float(_TOPK))
    out[...] = jnp.where(mask_fast, adj, 0.0)

    @pl.when(jnp.logical_not(ok))
    def _slow():
        col = jax.lax.broadcasted_iota(jnp.int32, adj.shape, 1)
        keys = sb | ((_N - 1) - col)
        k = keys
        m = None
        for _ in range(_TOPK):
            m = jnp.max(k, axis=1, keepdims=True)
            k = jnp.where(k == m, _INT_MIN, k)
        out[...] = jnp.where(keys >= m, adj, 0.0)


def kernel(idx, emb1_w, emb2_w, theta1_w, theta1_b, theta2_w, theta2_b):
    x1 = jnp.take(emb1_w, idx, axis=0)
    x2 = jnp.take(emb2_w, idx, axis=0)
    e1, e2 = pl.pallas_call(
        _embed_kernel,
        out_shape=[jax.ShapeDtypeStruct((_N, _D), jnp.float32)] * 2,
    )(x1, theta1_w, theta1_b.reshape(1, _D), x2, theta2_w, theta2_b.reshape(1, _D))

    grid = (_N // _BLOCK,)
    out = pl.pallas_call(
        _adj_kernel,
        grid=grid,
        in_specs=[
            pl.BlockSpec((_BLOCK, _D), lambda i: (i, 0)),
            pl.BlockSpec((_BLOCK, _D), lambda i: (i, 0)),
            pl.BlockSpec((_N, _D), lambda i: (0, 0)),
            pl.BlockSpec((_N, _D), lambda i: (0, 0)),
            pl.BlockSpec((_SEG, _SEG), lambda i: (0, 0)),
            pl.BlockSpec((_NSEG, _NSEG), lambda i: (0, 0)),
            pl.BlockSpec((_N, _NSEG), lambda i: (0, 0)),
        ],
        out_specs=pl.BlockSpec((_BLOCK, _N), lambda i: (i, 0)),
        out_shape=jax.ShapeDtypeStruct((_N, _N), jnp.float32),
    )(e1, e2, e1, e2, jnp.asarray(_UT128), jnp.asarray(_SUT32),
      jnp.asarray(_EMAT))
    return out
